# Initial kernel scaffold; baseline (speedup 1.0000x reference)
import jax
import jax.numpy as jnp
from jax.experimental import pallas as pl


def kernel(x, edge_index, adj_vals, eps, W1, W2, W3):
    raise NotImplementedError("write your pallas kernel here")


# speedup vs baseline: 4.5648x; 4.5648x over previous
import functools

import jax
import jax.numpy as jnp
from jax import lax
from jax.experimental import pallas as pl
from jax.experimental.pallas import tpu as pltpu
from jax.experimental.pallas import tpu_sc as plsc

N = 10000
E = 320000
D_IN, D_HID, D_LAT = 128, 64, 32
NC, NS, L = 2, 16, 16
NW = NC * NS
CHUNK = 128
CPT = 80
EPT = CPT * CHUNK
E_PAD = NW * EPT
OUT_PT = 624

_mesh = plsc.VectorSubcoreMesh(core_axis_name="c", subcore_axis_name="s")

_GATHER_DN = lax.GatherDimensionNumbers(
    offset_dims=(), collapsed_slice_dims=(0,), start_index_map=(0,))


def _lane_broadcast(vec, lane):
    idx = jnp.full((L, 1), lane, jnp.int32)
    return lax.gather(vec, idx, _GATHER_DN, (1,),
                      mode=lax.GatherScatterMode.PROMISE_IN_BOUNDS)


@functools.partial(
    pl.kernel,
    out_type=jax.ShapeDtypeStruct((NC, N, D_HID), jnp.float32),
    mesh=_mesh,
    scratch_types=[
        pltpu.VMEM((CPT, CHUNK), jnp.int32),
        pltpu.VMEM((CPT, CHUNK), jnp.int32),
        pltpu.VMEM((EPT,), jnp.float32),
        pltpu.VMEM((CHUNK, D_HID), jnp.float32),
        pltpu.VMEM_SHARED((N, D_HID), jnp.float32),
        pltpu.SemaphoreType.DMA,
    ],
    compiler_params=pltpu.CompilerParams(use_tc_tiling_on_sc=False),
)
def _spmm(table, srci, dsti, vals, out, idx_s, idx_d, vals_v, rows, acc, sem):
    c = lax.axis_index("c")
    s = lax.axis_index("s")
    wid = c * NS + s

    pltpu.sync_copy(srci.at[wid], idx_s)
    pltpu.sync_copy(dsti.at[wid], idx_d)
    pltpu.sync_copy(vals.at[wid], vals_v)

    def _zero_row(i, _):
        for k in range(D_HID // L):
            rows[i, pl.ds(L * k, L)] = jnp.zeros((L,), jnp.float32)
        return 0
    lax.fori_loop(0, CHUNK, _zero_row, 0)
    for k in range(5):
        pltpu.sync_copy(rows,
                        acc.at[pl.ds(s * OUT_PT + k * CHUNK, CHUNK)])
    plsc.subcore_barrier()

    def _chunk(j, _):
        pltpu.async_copy(table.at[idx_s.at[j]], rows, sem).wait()

        def _group(g, _):
            val16 = vals_v[pl.ds(j * CHUNK + g * L, L)]
            for e16 in range(L):
                v = _lane_broadcast(val16, e16)
                e = g * L + e16
                for k in range(D_HID // L):
                    sl = pl.ds(L * k, L)
                    rows[e, sl] = rows[e, sl] * v
            return 0
        lax.fori_loop(0, CHUNK // L, _group, 0)

        pltpu.sync_copy(rows, acc.at[idx_d.at[j]], add=True)
        return 0
    lax.fori_loop(0, CPT, _chunk, 0)
    plsc.subcore_barrier()

    pltpu.sync_copy(acc.at[pl.ds(s * OUT_PT, OUT_PT)],
                    out.at[c, pl.ds(s * OUT_PT, OUT_PT)])

    @pl.when(s == NS - 1)
    def _tail():
        pltpu.sync_copy(acc.at[pl.ds(NS * OUT_PT, N - NS * OUT_PT)],
                        out.at[c, pl.ds(NS * OUT_PT, N - NS * OUT_PT)])


def _mm_body(x_ref, w_ref, o_ref):
    o_ref[...] = jnp.dot(x_ref[...], w_ref[...],
                         preferred_element_type=jnp.float32)


def _relu_body(p_ref, o_ref):
    o_ref[...] = jnp.maximum(p_ref[0] + p_ref[1], 0.0)


def _final_body(p_ref, eps_ref, w2_ref, w3_ref, z_ref, mu_ref, lv_ref):
    ah = p_ref[0] + p_ref[1]
    mu = jnp.dot(ah, w2_ref[...], preferred_element_type=jnp.float32)
    lv = jnp.dot(ah, w3_ref[...], preferred_element_type=jnp.float32)
    z_ref[...] = eps_ref[...] * jnp.exp(lv) + mu
    mu_ref[...] = mu
    lv_ref[...] = lv


_BM = 1000


def kernel(x, edge_index, adj_vals, eps, W1, W2, W3):
    pad = E_PAD - E
    dst = jnp.concatenate([edge_index[0], jnp.zeros((pad,), jnp.int32)])
    src = jnp.concatenate([edge_index[1], jnp.zeros((pad,), jnp.int32)])
    vals = jnp.concatenate([adj_vals, jnp.zeros((pad,), jnp.float32)])
    dst3 = dst.reshape(NW, CPT, CHUNK)
    src3 = src.reshape(NW, CPT, CHUNK)
    vals2 = vals.reshape(NW, EPT)

    grid = N // _BM

    support = pl.pallas_call(
        _mm_body,
        grid=(grid,),
        in_specs=[pl.BlockSpec((_BM, D_IN), lambda i: (i, 0)),
                  pl.BlockSpec((D_IN, D_HID), lambda i: (0, 0))],
        out_specs=pl.BlockSpec((_BM, D_HID), lambda i: (i, 0)),
        out_shape=jax.ShapeDtypeStruct((N, D_HID), jnp.float32),
    )(x, W1)

    p1 = _spmm(support, src3, dst3, vals2)

    h1 = pl.pallas_call(
        _relu_body,
        grid=(grid,),
        in_specs=[pl.BlockSpec((NC, _BM, D_HID), lambda i: (0, i, 0))],
        out_specs=pl.BlockSpec((_BM, D_HID), lambda i: (i, 0)),
        out_shape=jax.ShapeDtypeStruct((N, D_HID), jnp.float32),
    )(p1)

    p2 = _spmm(h1, src3, dst3, vals2)

    z, mu, lv = pl.pallas_call(
        _final_body,
        grid=(grid,),
        in_specs=[pl.BlockSpec((NC, _BM, D_HID), lambda i: (0, i, 0)),
                  pl.BlockSpec((_BM, D_LAT), lambda i: (i, 0)),
                  pl.BlockSpec((D_HID, D_LAT), lambda i: (0, 0)),
                  pl.BlockSpec((D_HID, D_LAT), lambda i: (0, 0))],
        out_specs=[pl.BlockSpec((_BM, D_LAT), lambda i: (i, 0)),
                   pl.BlockSpec((_BM, D_LAT), lambda i: (i, 0)),
                   pl.BlockSpec((_BM, D_LAT), lambda i: (i, 0))],
        out_shape=[jax.ShapeDtypeStruct((N, D_LAT), jnp.float32),
                   jax.ShapeDtypeStruct((N, D_LAT), jnp.float32),
                   jax.ShapeDtypeStruct((N, D_LAT), jnp.float32)],
    )(p2, eps, W2, W3)

    return (z, mu, lv)


# speedup vs baseline: 7.1927x; 1.5757x over previous
import functools

import jax
import jax.numpy as jnp
from jax import lax
from jax.experimental import pallas as pl
from jax.experimental.pallas import tpu as pltpu
from jax.experimental.pallas import tpu_sc as plsc

N = 10000
E = 320000
D_IN, D_HID, D_LAT = 128, 64, 32
NC, NS, L = 2, 16, 16
NW = NC * NS
CHUNK = 128
CPT = 80
EPT = CPT * CHUNK
E_PAD = NW * EPT
OUT_PT = 624

_mesh = plsc.VectorSubcoreMesh(core_axis_name="c", subcore_axis_name="s")

_GATHER_DN = lax.GatherDimensionNumbers(
    offset_dims=(), collapsed_slice_dims=(0,), start_index_map=(0,))


def _lane_broadcast(vec, lane):
    idx = jnp.full((L, 1), lane, jnp.int32)
    return lax.gather(vec, idx, _GATHER_DN, (1,),
                      mode=lax.GatherScatterMode.PROMISE_IN_BOUNDS)


@functools.partial(
    pl.kernel,
    out_type=jax.ShapeDtypeStruct((NC, N, D_HID), jnp.float32),
    mesh=_mesh,
    scratch_types=[
        pltpu.VMEM((CPT, CHUNK), jnp.int32),
        pltpu.VMEM((CPT, CHUNK), jnp.int32),
        pltpu.VMEM((EPT,), jnp.float32),
        [pltpu.VMEM((CHUNK, D_HID), jnp.float32) for _ in range(4)],
        pltpu.VMEM_SHARED((N, D_HID), jnp.float32),
        [pltpu.SemaphoreType.DMA for _ in range(4)],
        [pltpu.SemaphoreType.DMA for _ in range(4)],
    ],
    compiler_params=pltpu.CompilerParams(use_tc_tiling_on_sc=False),
)
def _spmm(table, srci, dsti, vals, out, idx_s, idx_d, vals_v, rows, acc,
          gsem, ssem):
    c = lax.axis_index("c")
    s = lax.axis_index("s")
    wid = c * NS + s

    pltpu.sync_copy(srci.at[wid], idx_s)
    pltpu.sync_copy(dsti.at[wid], idx_d)
    pltpu.sync_copy(vals.at[wid], vals_v)

    def _zero_row(i, _):
        for k in range(D_HID // L):
            rows[0][i, pl.ds(L * k, L)] = jnp.zeros((L,), jnp.float32)
        return 0
    lax.fori_loop(0, CHUNK, _zero_row, 0)
    for k in range(5):
        pltpu.sync_copy(rows[0],
                        acc.at[pl.ds(s * OUT_PT + k * CHUNK, CHUNK)])
    plsc.subcore_barrier()

    def _fire_gather(j, b):
        pltpu.async_copy(table.at[idx_s.at[j]], rows[b], gsem[b])

    def _wait_gather(j, b):
        pltpu.make_async_copy(table.at[idx_s.at[j]], rows[b], gsem[b]).wait()

    def _fire_scatter(j, b):
        pltpu.async_copy(rows[b], acc.at[idx_d.at[j]], ssem[b], add=True)

    def _wait_scatter(j, b):
        pltpu.make_async_copy(rows[b], acc.at[idx_d.at[j]], ssem[b]).wait()

    def _multiply(j, b):
        def _group(g, _):
            val16 = vals_v[pl.ds(j * CHUNK + g * L, L)]
            for e16 in range(L):
                v = _lane_broadcast(val16, e16)
                e = g * L + e16
                for k in range(D_HID // L):
                    sl = pl.ds(L * k, L)
                    rows[b][e, sl] = rows[b][e, sl] * v
            return 0
        lax.fori_loop(0, CHUNK // L, _group, 0)

    for j in range(4):
        _fire_gather(j, j)
    for j in range(2):
        _wait_gather(j, j)
        _multiply(j, j)
        _fire_scatter(j, j)

    def _steady(J, _):
        jbase = 2 + J * 4
        for b in range(4):
            j = jbase + b
            bb = (2 + b) % 4
            _wait_gather(j, bb)
            _multiply(j, bb)
            _fire_scatter(j, bb)
            _wait_scatter(j - 2, b)
            _fire_gather(j + 2, b)
        return 0
    lax.fori_loop(0, (CPT - 4) // 4, _steady, 0)

    for j in range(CPT - 2, CPT):
        _wait_gather(j, j % 4)
        _multiply(j, j % 4)
        _fire_scatter(j, j % 4)
        _wait_scatter(j - 2, (j - 2) % 4)
    for j in range(CPT - 2, CPT):
        _wait_scatter(j, j % 4)

    plsc.subcore_barrier()

    pltpu.sync_copy(acc.at[pl.ds(s * OUT_PT, OUT_PT)],
                    out.at[c, pl.ds(s * OUT_PT, OUT_PT)])

    @pl.when(s == NS - 1)
    def _tail():
        pltpu.sync_copy(acc.at[pl.ds(NS * OUT_PT, N - NS * OUT_PT)],
                        out.at[c, pl.ds(NS * OUT_PT, N - NS * OUT_PT)])


def _mm_body(x_ref, w_ref, o_ref):
    o_ref[...] = jnp.dot(x_ref[...], w_ref[...],
                         preferred_element_type=jnp.float32)


def _relu_body(p_ref, o_ref):
    o_ref[...] = jnp.maximum(p_ref[0] + p_ref[1], 0.0)


def _final_body(p_ref, eps_ref, w2_ref, w3_ref, z_ref, mu_ref, lv_ref):
    ah = p_ref[0] + p_ref[1]
    mu = jnp.dot(ah, w2_ref[...], preferred_element_type=jnp.float32)
    lv = jnp.dot(ah, w3_ref[...], preferred_element_type=jnp.float32)
    z_ref[...] = eps_ref[...] * jnp.exp(lv) + mu
    mu_ref[...] = mu
    lv_ref[...] = lv


_BM = 1000


def kernel(x, edge_index, adj_vals, eps, W1, W2, W3):
    pad = E_PAD - E
    dst = jnp.concatenate([edge_index[0], jnp.zeros((pad,), jnp.int32)])
    src = jnp.concatenate([edge_index[1], jnp.zeros((pad,), jnp.int32)])
    vals = jnp.concatenate([adj_vals, jnp.zeros((pad,), jnp.float32)])
    dst3 = dst.reshape(NW, CPT, CHUNK)
    src3 = src.reshape(NW, CPT, CHUNK)
    vals2 = vals.reshape(NW, EPT)

    grid = N // _BM

    support = pl.pallas_call(
        _mm_body,
        grid=(grid,),
        in_specs=[pl.BlockSpec((_BM, D_IN), lambda i: (i, 0)),
                  pl.BlockSpec((D_IN, D_HID), lambda i: (0, 0))],
        out_specs=pl.BlockSpec((_BM, D_HID), lambda i: (i, 0)),
        out_shape=jax.ShapeDtypeStruct((N, D_HID), jnp.float32),
    )(x, W1)

    p1 = _spmm(support, src3, dst3, vals2)

    h1 = pl.pallas_call(
        _relu_body,
        grid=(grid,),
        in_specs=[pl.BlockSpec((NC, _BM, D_HID), lambda i: (0, i, 0))],
        out_specs=pl.BlockSpec((_BM, D_HID), lambda i: (i, 0)),
        out_shape=jax.ShapeDtypeStruct((N, D_HID), jnp.float32),
    )(p1)

    p2 = _spmm(h1, src3, dst3, vals2)

    z, mu, lv = pl.pallas_call(
        _final_body,
        grid=(grid,),
        in_specs=[pl.BlockSpec((NC, _BM, D_HID), lambda i: (0, i, 0)),
                  pl.BlockSpec((_BM, D_LAT), lambda i: (i, 0)),
                  pl.BlockSpec((D_HID, D_LAT), lambda i: (0, 0)),
                  pl.BlockSpec((D_HID, D_LAT), lambda i: (0, 0))],
        out_specs=[pl.BlockSpec((_BM, D_LAT), lambda i: (i, 0)),
                   pl.BlockSpec((_BM, D_LAT), lambda i: (i, 0)),
                   pl.BlockSpec((_BM, D_LAT), lambda i: (i, 0))],
        out_shape=[jax.ShapeDtypeStruct((N, D_LAT), jnp.float32),
                   jax.ShapeDtypeStruct((N, D_LAT), jnp.float32),
                   jax.ShapeDtypeStruct((N, D_LAT), jnp.float32)],
    )(p2, eps, W2, W3)

    return (z, mu, lv)


# speedup vs baseline: 9.3445x; 1.2992x over previous
import functools

import jax
import jax.numpy as jnp
from jax import lax
from jax.experimental import pallas as pl
from jax.experimental.pallas import tpu as pltpu
from jax.experimental.pallas import tpu_sc as plsc

N = 10000
E = 320000
D_IN, D_HID, D_LAT = 128, 64, 32
NC, NS, L = 2, 16, 16
NW = NC * NS
CHUNK = 128
CPT = 80
NPH = 4
CPP = CPT // NPH
EPT = CPT * CHUNK
E_PAD = NW * EPT
OUT_PT = 624

_mesh = plsc.VectorSubcoreMesh(core_axis_name="c", subcore_axis_name="s")

_GATHER_DN = lax.GatherDimensionNumbers(
    offset_dims=(), collapsed_slice_dims=(0,), start_index_map=(0,))


def _lane_broadcast(vec, lane):
    idx = jnp.full((L, 1), lane, jnp.int32)
    return lax.gather(vec, idx, _GATHER_DN, (1,),
                      mode=lax.GatherScatterMode.PROMISE_IN_BOUNDS)


@functools.partial(
    pl.kernel,
    out_type=jax.ShapeDtypeStruct((NC, N, D_HID), jnp.float32),
    mesh=_mesh,
    scratch_types=[
        pltpu.VMEM((CPP, CHUNK), jnp.int32),
        pltpu.VMEM((CPP, CHUNK), jnp.int32),
        pltpu.VMEM((CPP * CHUNK,), jnp.float32),
        [pltpu.VMEM((CHUNK, D_HID), jnp.float32) for _ in range(4)],
        pltpu.VMEM_SHARED((N, D_HID), jnp.float32),
        pltpu.VMEM_SHARED((N, D_HID), jnp.float32),
        [pltpu.SemaphoreType.DMA for _ in range(4)],
        [pltpu.SemaphoreType.DMA for _ in range(4)],
    ],
    compiler_params=pltpu.CompilerParams(use_tc_tiling_on_sc=False),
)
def _spmm(table, srci, dsti, vals, out, idx_s, idx_d, vals_v, rows, acc,
          tab, gsem, ssem):
    c = lax.axis_index("c")
    s = lax.axis_index("s")
    wid = c * NS + s

    pltpu.sync_copy(table.at[pl.ds(s * OUT_PT, N - (NS - 1) * OUT_PT)],
                    tab.at[pl.ds(s * OUT_PT, N - (NS - 1) * OUT_PT)])

    def _zero_row(i, _):
        for k in range(D_HID // L):
            rows[0][i, pl.ds(L * k, L)] = jnp.zeros((L,), jnp.float32)
        return 0
    lax.fori_loop(0, CHUNK, _zero_row, 0)
    for k in range(5):
        pltpu.sync_copy(rows[0],
                        acc.at[pl.ds(s * OUT_PT + k * CHUNK, CHUNK)])
    plsc.subcore_barrier()

    def _fire_gather(j, b):
        pltpu.async_copy(tab.at[idx_s.at[j]], rows[b], gsem[b])

    def _wait_gather(j, b):
        pltpu.make_async_copy(tab.at[idx_s.at[j]], rows[b], gsem[b]).wait()

    def _fire_scatter(j, b):
        pltpu.async_copy(rows[b], acc.at[idx_d.at[j]], ssem[b], add=True)

    def _wait_scatter(j, b):
        pltpu.make_async_copy(rows[b], acc.at[idx_d.at[j]], ssem[b]).wait()

    def _multiply(j, b):
        def _group(g, _):
            val16 = vals_v[pl.ds(j * CHUNK + g * L, L)]
            for e16 in range(L):
                v = _lane_broadcast(val16, e16)
                e = g * L + e16
                for k in range(D_HID // L):
                    sl = pl.ds(L * k, L)
                    rows[b][e, sl] = rows[b][e, sl] * v
            return 0
        lax.fori_loop(0, CHUNK // L, _group, 0)

    for p in range(NPH):
        pltpu.sync_copy(srci.at[wid, p], idx_s)
        pltpu.sync_copy(dsti.at[wid, p], idx_d)
        pltpu.sync_copy(vals.at[wid, p], vals_v)

        for j in range(4):
            _fire_gather(j, j)
        for j in range(2):
            _wait_gather(j, j)
            _multiply(j, j)
            _fire_scatter(j, j)

        def _steady(J, _):
            jbase = 2 + J * 4
            for b in range(4):
                j = jbase + b
                bb = (2 + b) % 4
                _wait_gather(j, bb)
                _multiply(j, bb)
                _fire_scatter(j, bb)
                _wait_scatter(j - 2, b)
                _fire_gather(j + 2, b)
            return 0
        lax.fori_loop(0, (CPP - 4) // 4, _steady, 0)

        for j in range(CPP - 2, CPP):
            _wait_gather(j, j % 4)
            _multiply(j, j % 4)
            _fire_scatter(j, j % 4)
            _wait_scatter(j - 2, (j - 2) % 4)
        for j in range(CPP - 2, CPP):
            _wait_scatter(j, j % 4)

    plsc.subcore_barrier()

    pltpu.sync_copy(acc.at[pl.ds(s * OUT_PT, OUT_PT)],
                    out.at[c, pl.ds(s * OUT_PT, OUT_PT)])

    @pl.when(s == NS - 1)
    def _tail():
        pltpu.sync_copy(acc.at[pl.ds(NS * OUT_PT, N - NS * OUT_PT)],
                        out.at[c, pl.ds(NS * OUT_PT, N - NS * OUT_PT)])


def _mm_body(x_ref, w_ref, o_ref):
    o_ref[...] = jnp.dot(x_ref[...], w_ref[...],
                         preferred_element_type=jnp.float32)


def _relu_body(p_ref, o_ref):
    o_ref[...] = jnp.maximum(p_ref[0] + p_ref[1], 0.0)


def _final_body(p_ref, eps_ref, w2_ref, w3_ref, z_ref, mu_ref, lv_ref):
    ah = p_ref[0] + p_ref[1]
    mu = jnp.dot(ah, w2_ref[...], preferred_element_type=jnp.float32)
    lv = jnp.dot(ah, w3_ref[...], preferred_element_type=jnp.float32)
    z_ref[...] = eps_ref[...] * jnp.exp(lv) + mu
    mu_ref[...] = mu
    lv_ref[...] = lv


_BM = 1000


def kernel(x, edge_index, adj_vals, eps, W1, W2, W3):
    pad = E_PAD - E
    dst = jnp.concatenate([edge_index[0], jnp.zeros((pad,), jnp.int32)])
    src = jnp.concatenate([edge_index[1], jnp.zeros((pad,), jnp.int32)])
    vals = jnp.concatenate([adj_vals, jnp.zeros((pad,), jnp.float32)])
    dst3 = dst.reshape(NW, NPH, CPP, CHUNK)
    src3 = src.reshape(NW, NPH, CPP, CHUNK)
    vals2 = vals.reshape(NW, NPH, CPP * CHUNK)

    grid = N // _BM

    support = pl.pallas_call(
        _mm_body,
        grid=(grid,),
        in_specs=[pl.BlockSpec((_BM, D_IN), lambda i: (i, 0)),
                  pl.BlockSpec((D_IN, D_HID), lambda i: (0, 0))],
        out_specs=pl.BlockSpec((_BM, D_HID), lambda i: (i, 0)),
        out_shape=jax.ShapeDtypeStruct((N, D_HID), jnp.float32),
    )(x, W1)

    p1 = _spmm(support, src3, dst3, vals2)

    h1 = pl.pallas_call(
        _relu_body,
        grid=(grid,),
        in_specs=[pl.BlockSpec((NC, _BM, D_HID), lambda i: (0, i, 0))],
        out_specs=pl.BlockSpec((_BM, D_HID), lambda i: (i, 0)),
        out_shape=jax.ShapeDtypeStruct((N, D_HID), jnp.float32),
    )(p1)

    p2 = _spmm(h1, src3, dst3, vals2)

    z, mu, lv = pl.pallas_call(
        _final_body,
        grid=(grid,),
        in_specs=[pl.BlockSpec((NC, _BM, D_HID), lambda i: (0, i, 0)),
                  pl.BlockSpec((_BM, D_LAT), lambda i: (i, 0)),
                  pl.BlockSpec((D_HID, D_LAT), lambda i: (0, 0)),
                  pl.BlockSpec((D_HID, D_LAT), lambda i: (0, 0))],
        out_specs=[pl.BlockSpec((_BM, D_LAT), lambda i: (i, 0)),
                   pl.BlockSpec((_BM, D_LAT), lambda i: (i, 0)),
                   pl.BlockSpec((_BM, D_LAT), lambda i: (i, 0))],
        out_shape=[jax.ShapeDtypeStruct((N, D_LAT), jnp.float32),
                   jax.ShapeDtypeStruct((N, D_LAT), jnp.float32),
                   jax.ShapeDtypeStruct((N, D_LAT), jnp.float32)],
    )(p2, eps, W2, W3)

    return (z, mu, lv)


# speedup vs baseline: 15.4914x; 1.6578x over previous
import functools

import jax
import jax.numpy as jnp
from jax import lax
from jax.experimental import pallas as pl
from jax.experimental.pallas import tpu as pltpu
from jax.experimental.pallas import tpu_sc as plsc

N = 10000
E = 320000
D_IN, D_HID, D_LAT = 128, 64, 32
NC, NS, L = 2, 16, 16
NW = NC * NS
CHUNK = 128
CPT = 80
NPH = 4
CPP = CPT // NPH
EPT = CPT * CHUNK
E_PAD = NW * EPT
OUT_PT = 624

_mesh = plsc.VectorSubcoreMesh(core_axis_name="c", subcore_axis_name="s")

_GATHER_DN = lax.GatherDimensionNumbers(
    offset_dims=(), collapsed_slice_dims=(0,), start_index_map=(0,))


def _lane_broadcast(vec, lane):
    idx = jnp.full((L, 1), lane, jnp.int32)
    return lax.gather(vec, idx, _GATHER_DN, (1,),
                      mode=lax.GatherScatterMode.PROMISE_IN_BOUNDS)


def _make_spmm(fused_relu_combine):
  @functools.partial(
      pl.kernel,
      out_type=jax.ShapeDtypeStruct((NC, N, D_HID), jnp.float32),
      mesh=_mesh,
      scratch_types=[
          pltpu.VMEM((CPP, CHUNK), jnp.int32),
          pltpu.VMEM((CPP, CHUNK), jnp.int32),
          pltpu.VMEM((CPP * CHUNK,), jnp.float32),
          [pltpu.VMEM((CHUNK, D_HID), jnp.float32) for _ in range(4)],
          pltpu.VMEM_SHARED((N, D_HID), jnp.float32),
          pltpu.VMEM_SHARED((N, D_HID), jnp.float32),
          [pltpu.SemaphoreType.DMA for _ in range(4)],
          [pltpu.SemaphoreType.DMA for _ in range(4)],
      ],
      compiler_params=pltpu.CompilerParams(use_tc_tiling_on_sc=False),
  )
  def _spmm(table, srci, dsti, vals, out, idx_s, idx_d, vals_v, rows, acc,
            tab, gsem, ssem):
    c = lax.axis_index("c")
    s = lax.axis_index("s")
    wid = c * NS + s

    if fused_relu_combine:
        for t in range(5):
            base = s * OUT_PT + t * CHUNK
            pltpu.sync_copy(table.at[0, pl.ds(base, CHUNK)], rows[0])
            pltpu.sync_copy(table.at[1, pl.ds(base, CHUNK)], rows[1])

            def _relu_row(i, _):
                for k in range(D_HID // L):
                    sl = pl.ds(L * k, L)
                    rows[0][i, sl] = jnp.maximum(
                        rows[0][i, sl] + rows[1][i, sl], 0.0)
                return 0
            lax.fori_loop(0, CHUNK, _relu_row, 0)
            pltpu.sync_copy(rows[0], tab.at[pl.ds(base, CHUNK)])
    else:
        pltpu.sync_copy(table.at[pl.ds(s * OUT_PT, N - (NS - 1) * OUT_PT)],
                        tab.at[pl.ds(s * OUT_PT, N - (NS - 1) * OUT_PT)])

    def _zero_row(i, _):
        for k in range(D_HID // L):
            rows[0][i, pl.ds(L * k, L)] = jnp.zeros((L,), jnp.float32)
        return 0
    lax.fori_loop(0, CHUNK, _zero_row, 0)
    for k in range(5):
        pltpu.sync_copy(rows[0],
                        acc.at[pl.ds(s * OUT_PT + k * CHUNK, CHUNK)])
    plsc.subcore_barrier()

    def _fire_gather(j, b):
        pltpu.async_copy(tab.at[idx_s.at[j]], rows[b], gsem[b])

    def _wait_gather(j, b):
        pltpu.make_async_copy(tab.at[idx_s.at[j]], rows[b], gsem[b]).wait()

    def _fire_scatter(j, b):
        pltpu.async_copy(rows[b], acc.at[idx_d.at[j]], ssem[b], add=True)

    def _wait_scatter(j, b):
        pltpu.make_async_copy(rows[b], acc.at[idx_d.at[j]], ssem[b]).wait()

    def _multiply(j, b):
        def _group(g, _):
            for gg in range(2):
                val16 = vals_v[pl.ds(j * CHUNK + (g * 2 + gg) * L, L)]
                for e16 in range(L):
                    v = _lane_broadcast(val16, e16)
                    e = (g * 2 + gg) * L + e16
                    for k in range(D_HID // L):
                        sl = pl.ds(L * k, L)
                        rows[b][e, sl] = rows[b][e, sl] * v
            return 0
        lax.fori_loop(0, CHUNK // L // 2, _group, 0)

    for p in range(NPH):
        pltpu.sync_copy(srci.at[wid, p], idx_s)
        pltpu.sync_copy(dsti.at[wid, p], idx_d)
        pltpu.sync_copy(vals.at[wid, p], vals_v)

        for j in range(4):
            _fire_gather(j, j)
        for j in range(2):
            _wait_gather(j, j)
            _multiply(j, j)
            _fire_scatter(j, j)

        def _steady(J, _):
            jbase = 2 + J * 4
            for b in range(4):
                j = jbase + b
                bb = (2 + b) % 4
                _wait_gather(j, bb)
                _multiply(j, bb)
                _fire_scatter(j, bb)
                _wait_scatter(j - 2, b)
                _fire_gather(j + 2, b)
            return 0
        lax.fori_loop(0, (CPP - 4) // 4, _steady, 0)

        for j in range(CPP - 2, CPP):
            _wait_gather(j, j % 4)
            _multiply(j, j % 4)
            _fire_scatter(j, j % 4)
            _wait_scatter(j - 2, (j - 2) % 4)
        for j in range(CPP - 2, CPP):
            _wait_scatter(j, j % 4)

    plsc.subcore_barrier()

    pltpu.sync_copy(acc.at[pl.ds(s * OUT_PT, OUT_PT)],
                    out.at[c, pl.ds(s * OUT_PT, OUT_PT)])

    @pl.when(s == NS - 1)
    def _tail():
        pltpu.sync_copy(acc.at[pl.ds(NS * OUT_PT, N - NS * OUT_PT)],
                        out.at[c, pl.ds(NS * OUT_PT, N - NS * OUT_PT)])

  return _spmm


_spmm_plain = _make_spmm(False)
_spmm_fused = _make_spmm(True)


def _mm_body(x_ref, w_ref, o_ref):
    o_ref[...] = jnp.dot(x_ref[...], w_ref[...],
                         preferred_element_type=jnp.float32)


def _final_body(p_ref, eps_ref, w2_ref, w3_ref, z_ref, mu_ref, lv_ref):
    ah = p_ref[0] + p_ref[1]
    mu = jnp.dot(ah, w2_ref[...], preferred_element_type=jnp.float32)
    lv = jnp.dot(ah, w3_ref[...], preferred_element_type=jnp.float32)
    z_ref[...] = eps_ref[...] * jnp.exp(lv) + mu
    mu_ref[...] = mu
    lv_ref[...] = lv


_BM = 1000


def kernel(x, edge_index, adj_vals, eps, W1, W2, W3):
    pad = E_PAD - E
    dst = jnp.concatenate([edge_index[0], jnp.zeros((pad,), jnp.int32)])
    src = jnp.concatenate([edge_index[1], jnp.zeros((pad,), jnp.int32)])
    vals = jnp.concatenate([adj_vals, jnp.zeros((pad,), jnp.float32)])
    dst3 = dst.reshape(NW, NPH, CPP, CHUNK)
    src3 = src.reshape(NW, NPH, CPP, CHUNK)
    vals2 = vals.reshape(NW, NPH, CPP * CHUNK)

    grid = N // _BM

    support = pl.pallas_call(
        _mm_body,
        grid=(grid,),
        in_specs=[pl.BlockSpec((_BM, D_IN), lambda i: (i, 0)),
                  pl.BlockSpec((D_IN, D_HID), lambda i: (0, 0))],
        out_specs=pl.BlockSpec((_BM, D_HID), lambda i: (i, 0)),
        out_shape=jax.ShapeDtypeStruct((N, D_HID), jnp.float32),
    )(x, W1)

    p1 = _spmm_plain(support, src3, dst3, vals2)
    p2 = _spmm_fused(p1, src3, dst3, vals2)

    z, mu, lv = pl.pallas_call(
        _final_body,
        grid=(grid,),
        in_specs=[pl.BlockSpec((NC, _BM, D_HID), lambda i: (0, i, 0)),
                  pl.BlockSpec((_BM, D_LAT), lambda i: (i, 0)),
                  pl.BlockSpec((D_HID, D_LAT), lambda i: (0, 0)),
                  pl.BlockSpec((D_HID, D_LAT), lambda i: (0, 0))],
        out_specs=[pl.BlockSpec((_BM, D_LAT), lambda i: (i, 0)),
                   pl.BlockSpec((_BM, D_LAT), lambda i: (i, 0)),
                   pl.BlockSpec((_BM, D_LAT), lambda i: (i, 0))],
        out_shape=[jax.ShapeDtypeStruct((N, D_LAT), jnp.float32),
                   jax.ShapeDtypeStruct((N, D_LAT), jnp.float32),
                   jax.ShapeDtypeStruct((N, D_LAT), jnp.float32)],
    )(p2, eps, W2, W3)

    return (z, mu, lv)


# speedup vs baseline: 16.3063x; 1.0526x over previous
import functools

import jax
import jax.numpy as jnp
from jax import lax
from jax.experimental import pallas as pl
from jax.experimental.pallas import tpu as pltpu
from jax.experimental.pallas import tpu_sc as plsc

N = 10000
E = 320000
D_IN, D_HID, D_LAT = 128, 64, 32
NC, NS, L = 2, 16, 16
NW = NC * NS
CHUNK = 128
CPT = 80
NPH = 2
CPP = CPT // NPH
EPT = CPT * CHUNK
E_PAD = NW * EPT
OUT_PT = 624

_mesh = plsc.VectorSubcoreMesh(core_axis_name="c", subcore_axis_name="s")

_GATHER_DN = lax.GatherDimensionNumbers(
    offset_dims=(), collapsed_slice_dims=(0,), start_index_map=(0,))


def _lane_broadcast(vec, lane):
    idx = jnp.full((L, 1), lane, jnp.int32)
    return lax.gather(vec, idx, _GATHER_DN, (1,),
                      mode=lax.GatherScatterMode.PROMISE_IN_BOUNDS)


def _make_spmm(fused_relu_combine):
  @functools.partial(
      pl.kernel,
      out_type=jax.ShapeDtypeStruct((NC, N, D_HID), jnp.float32),
      mesh=_mesh,
      scratch_types=[
          pltpu.VMEM((CPP, CHUNK), jnp.int32),
          pltpu.VMEM((CPP, CHUNK), jnp.int32),
          pltpu.VMEM((CPP * CHUNK,), jnp.float32),
          [pltpu.VMEM((CHUNK, D_HID), jnp.float32) for _ in range(4)],
          pltpu.VMEM_SHARED((N, D_HID), jnp.float32),
          pltpu.VMEM_SHARED((N, D_HID), jnp.float32),
          [pltpu.SemaphoreType.DMA for _ in range(4)],
          [pltpu.SemaphoreType.DMA for _ in range(4)],
      ],
      compiler_params=pltpu.CompilerParams(use_tc_tiling_on_sc=False),
  )
  def _spmm(table, srci, dsti, vals, out, idx_s, idx_d, vals_v, rows, acc,
            tab, gsem, ssem):
    c = lax.axis_index("c")
    s = lax.axis_index("s")
    wid = c * NS + s

    if fused_relu_combine:
        for t in range(5):
            base = s * OUT_PT + t * CHUNK
            pltpu.sync_copy(table.at[0, pl.ds(base, CHUNK)], rows[0])
            pltpu.sync_copy(table.at[1, pl.ds(base, CHUNK)], rows[1])

            def _relu_row(i, _):
                for k in range(D_HID // L):
                    sl = pl.ds(L * k, L)
                    rows[0][i, sl] = jnp.maximum(
                        rows[0][i, sl] + rows[1][i, sl], 0.0)
                return 0
            lax.fori_loop(0, CHUNK, _relu_row, 0)
            pltpu.sync_copy(rows[0], tab.at[pl.ds(base, CHUNK)])
    else:
        pltpu.sync_copy(table.at[pl.ds(s * OUT_PT, N - (NS - 1) * OUT_PT)],
                        tab.at[pl.ds(s * OUT_PT, N - (NS - 1) * OUT_PT)])

    def _zero_row(i, _):
        for k in range(D_HID // L):
            rows[0][i, pl.ds(L * k, L)] = jnp.zeros((L,), jnp.float32)
        return 0
    lax.fori_loop(0, CHUNK, _zero_row, 0)
    for k in range(5):
        pltpu.sync_copy(rows[0],
                        acc.at[pl.ds(s * OUT_PT + k * CHUNK, CHUNK)])
    plsc.subcore_barrier()

    def _fire_gather(j, b):
        pltpu.async_copy(tab.at[idx_s.at[j]], rows[b], gsem[b])

    def _wait_gather(j, b):
        pltpu.make_async_copy(tab.at[idx_s.at[j]], rows[b], gsem[b]).wait()

    def _fire_scatter(j, b):
        pltpu.async_copy(rows[b], acc.at[idx_d.at[j]], ssem[b], add=True)

    def _wait_scatter(j, b):
        pltpu.make_async_copy(rows[b], acc.at[idx_d.at[j]], ssem[b]).wait()

    def _multiply(j, b):
        def _group(g, _):
            for gg in range(4):
                val16 = vals_v[pl.ds(j * CHUNK + (g * 4 + gg) * L, L)]
                for e16 in range(L):
                    v = _lane_broadcast(val16, e16)
                    e = (g * 4 + gg) * L + e16
                    for k in range(D_HID // L):
                        sl = pl.ds(L * k, L)
                        rows[b][e, sl] = rows[b][e, sl] * v
            return 0
        lax.fori_loop(0, CHUNK // L // 4, _group, 0)

    for p in range(NPH):
        pltpu.sync_copy(srci.at[wid, p], idx_s)
        pltpu.sync_copy(dsti.at[wid, p], idx_d)
        pltpu.sync_copy(vals.at[wid, p], vals_v)

        for j in range(4):
            _fire_gather(j, j)
        for j in range(2):
            _wait_gather(j, j)
            _multiply(j, j)
            _fire_scatter(j, j)

        def _steady(J, _):
            jbase = 2 + J * 4
            for b in range(4):
                j = jbase + b
                bb = (2 + b) % 4
                _wait_gather(j, bb)
                _multiply(j, bb)
                _fire_scatter(j, bb)
                _wait_scatter(j - 2, b)
                _fire_gather(j + 2, b)
            return 0
        lax.fori_loop(0, (CPP - 4) // 4, _steady, 0)

        for j in range(CPP - 2, CPP):
            _wait_gather(j, j % 4)
            _multiply(j, j % 4)
            _fire_scatter(j, j % 4)
            _wait_scatter(j - 2, (j - 2) % 4)
        for j in range(CPP - 2, CPP):
            _wait_scatter(j, j % 4)

    plsc.subcore_barrier()

    pltpu.sync_copy(acc.at[pl.ds(s * OUT_PT, OUT_PT)],
                    out.at[c, pl.ds(s * OUT_PT, OUT_PT)])

    @pl.when(s == NS - 1)
    def _tail():
        pltpu.sync_copy(acc.at[pl.ds(NS * OUT_PT, N - NS * OUT_PT)],
                        out.at[c, pl.ds(NS * OUT_PT, N - NS * OUT_PT)])

  return _spmm


_spmm_plain = _make_spmm(False)
_spmm_fused = _make_spmm(True)


def _mm_body(x_ref, w_ref, o_ref):
    o_ref[...] = jnp.dot(x_ref[...], w_ref[...],
                         preferred_element_type=jnp.float32)


def _final_body(p_ref, eps_ref, w2_ref, w3_ref, z_ref, mu_ref, lv_ref):
    ah = p_ref[0] + p_ref[1]
    mu = jnp.dot(ah, w2_ref[...], preferred_element_type=jnp.float32)
    lv = jnp.dot(ah, w3_ref[...], preferred_element_type=jnp.float32)
    z_ref[...] = eps_ref[...] * jnp.exp(lv) + mu
    mu_ref[...] = mu
    lv_ref[...] = lv


_BM = 1000


def kernel(x, edge_index, adj_vals, eps, W1, W2, W3):
    pad = E_PAD - E
    dst = jnp.concatenate([edge_index[0], jnp.zeros((pad,), jnp.int32)])
    src = jnp.concatenate([edge_index[1], jnp.zeros((pad,), jnp.int32)])
    vals = jnp.concatenate([adj_vals, jnp.zeros((pad,), jnp.float32)])
    dst3 = dst.reshape(NW, NPH, CPP, CHUNK)
    src3 = src.reshape(NW, NPH, CPP, CHUNK)
    vals2 = vals.reshape(NW, NPH, CPP * CHUNK)

    grid = N // _BM

    support = pl.pallas_call(
        _mm_body,
        grid=(grid,),
        in_specs=[pl.BlockSpec((_BM, D_IN), lambda i: (i, 0)),
                  pl.BlockSpec((D_IN, D_HID), lambda i: (0, 0))],
        out_specs=pl.BlockSpec((_BM, D_HID), lambda i: (i, 0)),
        out_shape=jax.ShapeDtypeStruct((N, D_HID), jnp.float32),
    )(x, W1)

    p1 = _spmm_plain(support, src3, dst3, vals2)
    p2 = _spmm_fused(p1, src3, dst3, vals2)

    z, mu, lv = pl.pallas_call(
        _final_body,
        grid=(grid,),
        in_specs=[pl.BlockSpec((NC, _BM, D_HID), lambda i: (0, i, 0)),
                  pl.BlockSpec((_BM, D_LAT), lambda i: (i, 0)),
                  pl.BlockSpec((D_HID, D_LAT), lambda i: (0, 0)),
                  pl.BlockSpec((D_HID, D_LAT), lambda i: (0, 0))],
        out_specs=[pl.BlockSpec((_BM, D_LAT), lambda i: (i, 0)),
                   pl.BlockSpec((_BM, D_LAT), lambda i: (i, 0)),
                   pl.BlockSpec((_BM, D_LAT), lambda i: (i, 0))],
        out_shape=[jax.ShapeDtypeStruct((N, D_LAT), jnp.float32),
                   jax.ShapeDtypeStruct((N, D_LAT), jnp.float32),
                   jax.ShapeDtypeStruct((N, D_LAT), jnp.float32)],
    )(p2, eps, W2, W3)

    return (z, mu, lv)


# speedup vs baseline: 16.7320x; 1.0261x over previous
import functools

import jax
import jax.numpy as jnp
from jax import lax
from jax.experimental import pallas as pl
from jax.experimental.pallas import tpu as pltpu
from jax.experimental.pallas import tpu_sc as plsc

N = 10000
E = 320000
D_IN, D_HID, D_LAT = 128, 64, 32
NC, NS, L = 2, 16, 16
NW = NC * NS
CHUNK = 128
CPT = 80
NPH = 2
CPP = CPT // NPH
EPT = CPT * CHUNK
E_PAD = NW * EPT
OUT_PT = 624

_mesh = plsc.VectorSubcoreMesh(core_axis_name="c", subcore_axis_name="s")

_GATHER_DN = lax.GatherDimensionNumbers(
    offset_dims=(), collapsed_slice_dims=(0,), start_index_map=(0,))


def _lane_broadcast(vec, lane):
    idx = jnp.full((L, 1), lane, jnp.int32)
    return lax.gather(vec, idx, _GATHER_DN, (1,),
                      mode=lax.GatherScatterMode.PROMISE_IN_BOUNDS)


def _make_spmm(fused_relu_combine):
  @functools.partial(
      pl.kernel,
      out_type=jax.ShapeDtypeStruct((NC, N, D_HID), jnp.float32),
      mesh=_mesh,
      scratch_types=[
          pltpu.VMEM((CPP, CHUNK), jnp.int32),
          pltpu.VMEM((CPP, CHUNK), jnp.int32),
          pltpu.VMEM((CPP * CHUNK,), jnp.float32),
          [pltpu.VMEM((CHUNK, D_HID), jnp.float32) for _ in range(4)],
          pltpu.VMEM_SHARED((N, D_HID), jnp.float32),
          pltpu.VMEM_SHARED((N, D_HID), jnp.float32),
          [pltpu.SemaphoreType.DMA for _ in range(4)],
          [pltpu.SemaphoreType.DMA for _ in range(4)],
      ],
      compiler_params=pltpu.CompilerParams(use_tc_tiling_on_sc=False),
  )
  def _spmm(table, srci, dsti, vals, out, idx_s, idx_d, vals_v, rows, acc,
            tab, gsem, ssem):
    c = lax.axis_index("c")
    s = lax.axis_index("s")
    wid = c * NS + s

    if fused_relu_combine:
        def _fire_stage(t):
            pa = (t % 2) * 2
            base = s * OUT_PT + t * CHUNK
            pltpu.async_copy(table.at[0, pl.ds(base, CHUNK)],
                             rows[pa], gsem[pa])
            pltpu.async_copy(table.at[1, pl.ds(base, CHUNK)],
                             rows[pa + 1], gsem[pa + 1])

        def _wait_stage(t):
            pa = (t % 2) * 2
            base = s * OUT_PT + t * CHUNK
            pltpu.make_async_copy(table.at[0, pl.ds(base, CHUNK)],
                                  rows[pa], gsem[pa]).wait()
            pltpu.make_async_copy(table.at[1, pl.ds(base, CHUNK)],
                                  rows[pa + 1], gsem[pa + 1]).wait()

        _fire_stage(0)
        for t in range(5):
            if t < 4:
                _fire_stage(t + 1)
            _wait_stage(t)
            pa = (t % 2) * 2
            base = s * OUT_PT + t * CHUNK

            def _relu_row(i, _):
                for k in range(D_HID // L):
                    sl = pl.ds(L * k, L)
                    rows[pa][i, sl] = jnp.maximum(
                        rows[pa][i, sl] + rows[pa + 1][i, sl], 0.0)
                return 0
            lax.fori_loop(0, CHUNK, _relu_row, 0)
            pltpu.sync_copy(rows[pa], tab.at[pl.ds(base, CHUNK)])
    else:
        pltpu.sync_copy(table.at[pl.ds(s * OUT_PT, N - (NS - 1) * OUT_PT)],
                        tab.at[pl.ds(s * OUT_PT, N - (NS - 1) * OUT_PT)])

    def _zero_row(i, _):
        for k in range(D_HID // L):
            rows[0][i, pl.ds(L * k, L)] = jnp.zeros((L,), jnp.float32)
        return 0
    lax.fori_loop(0, CHUNK, _zero_row, 0)
    for k in range(5):
        pltpu.sync_copy(rows[0],
                        acc.at[pl.ds(s * OUT_PT + k * CHUNK, CHUNK)])
    plsc.subcore_barrier()

    def _fire_gather(j, b):
        pltpu.async_copy(tab.at[idx_s.at[j]], rows[b], gsem[b])

    def _wait_gather(j, b):
        pltpu.make_async_copy(tab.at[idx_s.at[j]], rows[b], gsem[b]).wait()

    def _fire_scatter(j, b):
        pltpu.async_copy(rows[b], acc.at[idx_d.at[j]], ssem[b], add=True)

    def _wait_scatter(j, b):
        pltpu.make_async_copy(rows[b], acc.at[idx_d.at[j]], ssem[b]).wait()

    def _multiply(j, b):
        def _group(g, _):
            for gg in range(4):
                val16 = vals_v[pl.ds(j * CHUNK + (g * 4 + gg) * L, L)]
                for e16 in range(L):
                    v = _lane_broadcast(val16, e16)
                    e = (g * 4 + gg) * L + e16
                    for k in range(D_HID // L):
                        sl = pl.ds(L * k, L)
                        rows[b][e, sl] = rows[b][e, sl] * v
            return 0
        lax.fori_loop(0, CHUNK // L // 4, _group, 0)

    for p in range(NPH):
        pltpu.sync_copy(srci.at[wid, p], idx_s)
        pltpu.sync_copy(dsti.at[wid, p], idx_d)
        pltpu.sync_copy(vals.at[wid, p], vals_v)

        for j in range(4):
            _fire_gather(j, j)
        for j in range(2):
            _wait_gather(j, j)
            _multiply(j, j)
            _fire_scatter(j, j)

        def _steady(J, _):
            jbase = 2 + J * 4
            for b in range(4):
                j = jbase + b
                bb = (2 + b) % 4
                _wait_gather(j, bb)
                _multiply(j, bb)
                _fire_scatter(j, bb)
                _wait_scatter(j - 2, b)
                _fire_gather(j + 2, b)
            return 0
        lax.fori_loop(0, (CPP - 4) // 4, _steady, 0)

        for j in range(CPP - 2, CPP):
            _wait_gather(j, j % 4)
            _multiply(j, j % 4)
            _fire_scatter(j, j % 4)
            _wait_scatter(j - 2, (j - 2) % 4)
        for j in range(CPP - 2, CPP):
            _wait_scatter(j, j % 4)

    plsc.subcore_barrier()

    pltpu.sync_copy(acc.at[pl.ds(s * OUT_PT, OUT_PT)],
                    out.at[c, pl.ds(s * OUT_PT, OUT_PT)])

    @pl.when(s == NS - 1)
    def _tail():
        pltpu.sync_copy(acc.at[pl.ds(NS * OUT_PT, N - NS * OUT_PT)],
                        out.at[c, pl.ds(NS * OUT_PT, N - NS * OUT_PT)])

  return _spmm


_spmm_plain = _make_spmm(False)
_spmm_fused = _make_spmm(True)


def _mm_body(x_ref, w_ref, o_ref):
    o_ref[...] = jnp.dot(x_ref[...], w_ref[...],
                         preferred_element_type=jnp.float32)


def _final_body(p_ref, eps_ref, w2_ref, w3_ref, z_ref, mu_ref, lv_ref):
    ah = p_ref[0] + p_ref[1]
    mu = jnp.dot(ah, w2_ref[...], preferred_element_type=jnp.float32)
    lv = jnp.dot(ah, w3_ref[...], preferred_element_type=jnp.float32)
    z_ref[...] = eps_ref[...] * jnp.exp(lv) + mu
    mu_ref[...] = mu
    lv_ref[...] = lv


_BM = 1000


def kernel(x, edge_index, adj_vals, eps, W1, W2, W3):
    pad = E_PAD - E
    dst = jnp.concatenate([edge_index[0], jnp.zeros((pad,), jnp.int32)])
    src = jnp.concatenate([edge_index[1], jnp.zeros((pad,), jnp.int32)])
    vals = jnp.concatenate([adj_vals, jnp.zeros((pad,), jnp.float32)])
    dst3 = dst.reshape(NW, NPH, CPP, CHUNK)
    src3 = src.reshape(NW, NPH, CPP, CHUNK)
    vals2 = vals.reshape(NW, NPH, CPP * CHUNK)

    grid = N // _BM

    support = pl.pallas_call(
        _mm_body,
        grid=(grid,),
        in_specs=[pl.BlockSpec((_BM, D_IN), lambda i: (i, 0)),
                  pl.BlockSpec((D_IN, D_HID), lambda i: (0, 0))],
        out_specs=pl.BlockSpec((_BM, D_HID), lambda i: (i, 0)),
        out_shape=jax.ShapeDtypeStruct((N, D_HID), jnp.float32),
    )(x, W1)

    p1 = _spmm_plain(support, src3, dst3, vals2)
    p2 = _spmm_fused(p1, src3, dst3, vals2)

    z, mu, lv = pl.pallas_call(
        _final_body,
        grid=(grid,),
        in_specs=[pl.BlockSpec((NC, _BM, D_HID), lambda i: (0, i, 0)),
                  pl.BlockSpec((_BM, D_LAT), lambda i: (i, 0)),
                  pl.BlockSpec((D_HID, D_LAT), lambda i: (0, 0)),
                  pl.BlockSpec((D_HID, D_LAT), lambda i: (0, 0))],
        out_specs=[pl.BlockSpec((_BM, D_LAT), lambda i: (i, 0)),
                   pl.BlockSpec((_BM, D_LAT), lambda i: (i, 0)),
                   pl.BlockSpec((_BM, D_LAT), lambda i: (i, 0))],
        out_shape=[jax.ShapeDtypeStruct((N, D_LAT), jnp.float32),
                   jax.ShapeDtypeStruct((N, D_LAT), jnp.float32),
                   jax.ShapeDtypeStruct((N, D_LAT), jnp.float32)],
    )(p2, eps, W2, W3)

    return (z, mu, lv)
